# baseline (speedup 1.0000x reference)
import functools

import jax
import jax.numpy as jnp
from jax import lax
from jax.experimental import pallas as pl
from jax.experimental.pallas import tpu as pltpu
from jax.experimental.pallas import tpu_sc as plsc

_N_E = 8192
_E_DIM = 256
_BETA = 0.25
_M = 9216
_BM = 256
_MT = _M // _BM


def _vq_body(z_ref, e_ref, idx_ref, oh_ref, loss_ref, ppl_ref,
             counts_acc, loss_acc):
    i = pl.program_id(0)
    z = z_ref[...]
    e = e_ref[...]
    zn = jnp.sum(z * z, axis=1, keepdims=True)
    en = jnp.sum(e * e, axis=1, keepdims=True).T
    mm = lax.dot_general(z, e, (((1,), (1,)), ((), ())),
                         preferred_element_type=jnp.float32)
    d = (zn + en) - 2.0 * mm
    dmin = jnp.min(d, axis=1, keepdims=True)
    col = lax.broadcasted_iota(jnp.int32, d.shape, 1)
    idx = jnp.min(jnp.where(d == dmin, col, _N_E), axis=1, keepdims=True)
    oh = (col == idx).astype(jnp.float32)
    oh_ref[...] = oh
    idx_ref[...] = idx

    part = jnp.sum(oh, axis=0, keepdims=True)
    psum = jnp.sum(dmin, axis=0, keepdims=True)

    @pl.when(i == 0)
    def _():
        counts_acc[...] = part
        loss_acc[...] = psum

    @pl.when(i > 0)
    def _():
        counts_acc[...] = counts_acc[...] + part
        loss_acc[...] = loss_acc[...] + psum

    @pl.when(i == _MT - 1)
    def _():
        loss_ref[...] = loss_acc[...] * ((1.0 + _BETA) / (_M * _E_DIM))
        e_mean = counts_acc[...] / jnp.float32(_M)
        ent = jnp.sum(e_mean * jnp.log(e_mean + 1e-10), axis=1, keepdims=True)
        ppl_ref[...] = jnp.exp(-ent)


_vq_call = pl.pallas_call(
    _vq_body,
    grid=(_MT,),
    in_specs=[
        pl.BlockSpec((_BM, _E_DIM), lambda i: (i, 0)),
        pl.BlockSpec((_N_E, _E_DIM), lambda i: (0, 0)),
    ],
    out_specs=[
        pl.BlockSpec((_BM, 1), lambda i: (i, 0)),
        pl.BlockSpec((_BM, _N_E), lambda i: (i, 0)),
        pl.BlockSpec((1, 1), lambda i: (0, 0)),
        pl.BlockSpec((1, 1), lambda i: (0, 0)),
    ],
    out_shape=[
        jax.ShapeDtypeStruct((_M, 1), jnp.int32),
        jax.ShapeDtypeStruct((_M, _N_E), jnp.float32),
        jax.ShapeDtypeStruct((1, 1), jnp.float32),
        jax.ShapeDtypeStruct((1, 1), jnp.float32),
    ],
    scratch_shapes=[
        pltpu.VMEM((1, _N_E), jnp.float32),
        pltpu.VMEM((1, 1), jnp.float32),
    ],
)


_NC = 2
_NS = 16
_NW = _NC * _NS
_BPW = _M // _NW


@functools.partial(
    pl.kernel,
    mesh=plsc.VectorSubcoreMesh(core_axis_name="c", subcore_axis_name="s"),
    out_type=jax.ShapeDtypeStruct((_M, _E_DIM), jnp.float32),
    scratch_types=[
        pltpu.VMEM((_BPW,), jnp.int32),
        pltpu.VMEM((_BPW, _E_DIM), jnp.float32),
        pltpu.SemaphoreType.DMA,
    ],
)
def _sc_gather(table_hbm, idx_hbm, out_hbm, idx_v, rows_v, sem):
    wid = lax.axis_index("s") * _NC + lax.axis_index("c")
    base = wid * _BPW
    pltpu.sync_copy(idx_hbm.at[pl.ds(base, _BPW)], idx_v)
    pltpu.async_copy(table_hbm.at[idx_v], rows_v, sem).wait()
    pltpu.sync_copy(rows_v, out_hbm.at[pl.ds(base, _BPW)])


def kernel(z, embedding_weight):
    zf = z.reshape(-1, _E_DIM)
    idx, oh, loss, ppl = _vq_call(zf, embedding_weight)
    zq = _sc_gather(embedding_weight, idx.reshape(_M))
    return (loss[0, 0], zq.reshape(z.shape), ppl[0, 0], oh, idx)


# speedup vs baseline: 1.4188x; 1.4188x over previous
import functools

import jax
import jax.numpy as jnp
from jax import lax
from jax.experimental import pallas as pl
from jax.experimental.pallas import tpu as pltpu
from jax.experimental.pallas import tpu_sc as plsc

_N_E = 8192
_E_DIM = 256
_BETA = 0.25
_M = 9216
_BM = 384
_MT = _M // _BM


def _vq_body(z_ref, e_ref, idx_ref, oh_ref, loss_ref, ppl_ref,
             counts_acc, loss_acc, en_acc):
    i = pl.program_id(0)
    z = z_ref[...]
    e = e_ref[...]
    zn = jnp.sum(z * z, axis=1, keepdims=True)

    @pl.when(i == 0)
    def _():
        en_acc[...] = jnp.sum(e * e, axis=1, keepdims=True).T

    en = en_acc[...]
    mm = lax.dot_general(z, e, (((1,), (1,)), ((), ())),
                         preferred_element_type=jnp.float32)
    d = (zn + en) - 2.0 * mm
    dmin = jnp.min(d, axis=1, keepdims=True)
    col = lax.broadcasted_iota(jnp.int32, d.shape, 1)
    idx = jnp.min(jnp.where(d == dmin, col, _N_E), axis=1, keepdims=True)
    oh = (col == idx).astype(jnp.float32)
    oh_ref[...] = oh
    idx_ref[...] = idx

    part = lax.dot_general(jnp.ones((1, _BM), jnp.float32), oh,
                           (((1,), (0,)), ((), ())),
                           preferred_element_type=jnp.float32)
    psum = jnp.sum(dmin, axis=0, keepdims=True)

    @pl.when(i == 0)
    def _():
        counts_acc[...] = part
        loss_acc[...] = psum

    @pl.when(i > 0)
    def _():
        counts_acc[...] = counts_acc[...] + part
        loss_acc[...] = loss_acc[...] + psum

    @pl.when(i == _MT - 1)
    def _():
        loss_ref[...] = loss_acc[...] * ((1.0 + _BETA) / (_M * _E_DIM))
        e_mean = counts_acc[...] / jnp.float32(_M)
        ent = jnp.sum(e_mean * jnp.log(e_mean + 1e-10), axis=1, keepdims=True)
        ppl_ref[...] = jnp.exp(-ent)


_vq_call = pl.pallas_call(
    _vq_body,
    grid=(_MT,),
    in_specs=[
        pl.BlockSpec((_BM, _E_DIM), lambda i: (i, 0)),
        pl.BlockSpec((_N_E, _E_DIM), lambda i: (0, 0)),
    ],
    out_specs=[
        pl.BlockSpec((_BM, 1), lambda i: (i, 0)),
        pl.BlockSpec((_BM, _N_E), lambda i: (i, 0)),
        pl.BlockSpec((1, 1), lambda i: (0, 0)),
        pl.BlockSpec((1, 1), lambda i: (0, 0)),
    ],
    out_shape=[
        jax.ShapeDtypeStruct((_M, 1), jnp.int32),
        jax.ShapeDtypeStruct((_M, _N_E), jnp.float32),
        jax.ShapeDtypeStruct((1, 1), jnp.float32),
        jax.ShapeDtypeStruct((1, 1), jnp.float32),
    ],
    scratch_shapes=[
        pltpu.VMEM((1, _N_E), jnp.float32),
        pltpu.VMEM((1, 1), jnp.float32),
        pltpu.VMEM((1, _N_E), jnp.float32),
    ],
)


_NC = 2
_NS = 16
_NW = _NC * _NS
_BPW = _M // _NW


@functools.partial(
    pl.kernel,
    mesh=plsc.VectorSubcoreMesh(core_axis_name="c", subcore_axis_name="s"),
    out_type=jax.ShapeDtypeStruct((_M, _E_DIM), jnp.float32),
    scratch_types=[
        pltpu.VMEM((_BPW,), jnp.int32),
        pltpu.VMEM((_BPW, _E_DIM), jnp.float32),
        pltpu.SemaphoreType.DMA,
    ],
)
def _sc_gather(table_hbm, idx_hbm, out_hbm, idx_v, rows_v, sem):
    wid = lax.axis_index("s") * _NC + lax.axis_index("c")
    base = wid * _BPW
    pltpu.sync_copy(idx_hbm.at[pl.ds(base, _BPW)], idx_v)
    pltpu.async_copy(table_hbm.at[idx_v], rows_v, sem).wait()
    pltpu.sync_copy(rows_v, out_hbm.at[pl.ds(base, _BPW)])


def kernel(z, embedding_weight):
    zf = z.reshape(-1, _E_DIM)
    idx, oh, loss, ppl = _vq_call(zf, embedding_weight)
    zq = _sc_gather(embedding_weight, idx.reshape(_M))
    return (loss[0, 0], zq.reshape(z.shape), ppl[0, 0], oh, idx)
